# baseline (speedup 1.0000x reference)
import functools

import jax
import jax.numpy as jnp
from jax.experimental import pallas as pl
from jax.experimental.pallas import tpu as pltpu

N_ROWS = 131072
N_TERMS = 64
BLOCK_ROWS = 8192
GRID = N_ROWS // BLOCK_ROWS


def _gj_body(k, carry):
    a, b = carry
    is_k_row = jax.lax.broadcasted_iota(jnp.int32, (N_TERMS, 1), 0) == k
    is_k_col = jax.lax.broadcasted_iota(jnp.int32, (1, N_TERMS), 1) == k
    row_k = jnp.sum(jnp.where(is_k_row, a, 0.0), axis=0, keepdims=True)
    pivot = jnp.sum(jnp.where(is_k_col, row_k, 0.0))
    inv_p = 1.0 / pivot
    norm_row = row_k * inv_p
    b_k = jnp.sum(jnp.where(is_k_row, b, 0.0)) * inv_p
    col = jnp.sum(jnp.where(is_k_col, a, 0.0), axis=1, keepdims=True)
    new_a = jnp.where(is_k_row, norm_row, a - col * norm_row)
    new_b = jnp.where(is_k_row, b_k, b - col * b_k)
    return new_a, new_b


def _fit_kernel(td_ref, theta_ref, out_ref, gram_ref, rhs_ref):
    i = pl.program_id(0)
    th = theta_ref[...]
    part_g = jax.lax.dot_general(
        th, th, (((0,), (0,)), ((), ())), preferred_element_type=jnp.float32)
    part_r = jax.lax.dot_general(
        th, td_ref[...], (((0,), (0,)), ((), ())),
        preferred_element_type=jnp.float32)

    @pl.when(i == 0)
    def _():
        gram_ref[...] = part_g
        rhs_ref[...] = part_r

    @pl.when(i > 0)
    def _():
        gram_ref[...] += part_g
        rhs_ref[...] += part_r

    @pl.when(i == GRID - 1)
    def _():
        a, b = jax.lax.fori_loop(
            0, N_TERMS, _gj_body, (gram_ref[...], rhs_ref[...]))
        out_ref[...] = b


@functools.partial(jax.jit, static_argnames=())
def kernel(time_derivs, thetas):
    return pl.pallas_call(
        _fit_kernel,
        grid=(GRID,),
        in_specs=[
            pl.BlockSpec((BLOCK_ROWS, 1), lambda i: (i, 0)),
            pl.BlockSpec((BLOCK_ROWS, N_TERMS), lambda i: (i, 0)),
        ],
        out_specs=pl.BlockSpec((N_TERMS, 1), lambda i: (0, 0)),
        out_shape=jax.ShapeDtypeStruct((N_TERMS, 1), jnp.float32),
        scratch_shapes=[
            pltpu.VMEM((N_TERMS, N_TERMS), jnp.float32),
            pltpu.VMEM((N_TERMS, 1), jnp.float32),
        ],
    )(time_derivs, thetas)
